# baseline (speedup 1.0000x reference)
import functools

import jax
import jax.numpy as jnp
from jax import lax
from jax.experimental import pallas as pl
from jax.experimental.pallas import tpu as pltpu
from jax.experimental.pallas import tpu_sc as plsc

N = 100000
E = 3200000
HID = 16

NPAD = 102400
NP8 = NPAD // 8
NREP = NPAD * HID

NC = 2
NS = 16
NW = NC * NS
G = 128
EROWS = 25600
KJ = 4
NB = 4
RB = 2
RC0 = 496
RC1 = 1104
O0 = RC0 // KJ
O1 = RC1 // KJ

_mesh = plsc.VectorSubcoreMesh(core_axis_name="c", subcore_axis_name="s")
_sc_params = pltpu.CompilerParams(use_tc_tiling_on_sc=False)


def _fill_rows(buf, nrows, vec):
  def body(i, _):
    buf[i] = vec
    return 0
  lax.fori_loop(0, nrows, body, 0)


def _zero_acc(acc, sid, zrows):
  chunk = NPAD // NS
  def zb(i, _):
    pltpu.sync_copy(zrows, acc.at[pl.ds(sid * chunk + i * 128, 128)])
    return 0
  lax.fori_loop(0, chunk // 128, zb, 0)


def _copy_out(acc, out_hbm, cid, sid):
  chunk = NPAD // NS
  off = sid * chunk
  pltpu.sync_copy(acc.at[pl.ds(off, chunk)], out_hbm.at[cid, pl.ds(off, chunk)])


@functools.partial(
    pl.kernel,
    mesh=_mesh,
    compiler_params=_sc_params,
    out_type=jax.ShapeDtypeStruct((NC, NPAD, HID), jnp.float32),
    scratch_types=[
        pltpu.VMEM((NB, KJ, G), jnp.int32),
        pltpu.VMEM((NB, KJ, G), jnp.int32),
        pltpu.VMEM((G, HID), jnp.float32),
        pltpu.VMEM((G, HID), jnp.float32),
        pltpu.VMEM((128, HID), jnp.float32),
        pltpu.VMEM_SHARED((NPAD, HID), jnp.float32),
        pltpu.SemaphoreType.DMA,
        pltpu.SemaphoreType.DMA,
    ],
)
def _sc_degrees(src_hbm, dst_hbm, out_hbm, idxS, idxD, e0buf, e1buf, zrows, acc,
                sem_i, sem_s):
  cid = lax.axis_index("c")
  sid = lax.axis_index("s")

  lane = lax.iota(jnp.int32, HID)
  zvec = jnp.zeros((HID,), jnp.float32)
  e0 = jnp.where(lane == 0, 1.0, 0.0).astype(jnp.float32)
  e1 = jnp.where(lane == 1, 1.0, 0.0).astype(jnp.float32)
  _fill_rows(e0buf, G, e0)
  _fill_rows(e1buf, G, e1)
  _fill_rows(zrows, 128, zvec)

  _zero_acc(acc, sid, zrows)
  plsc.subcore_barrier()

  def pipeline(r0, outer):
    def idx_copies(g, b):
      r = r0 + g * KJ
      return (pltpu.make_async_copy(src_hbm.at[pl.ds(r, KJ)], idxS.at[b], sem_i),
              pltpu.make_async_copy(dst_hbm.at[pl.ds(r, KJ)], idxD.at[b], sem_i))

    def scat_copies(b, j):
      return (pltpu.make_async_copy(e0buf, acc.at[idxS.at[b, j]], sem_s),
              pltpu.make_async_copy(e1buf, acc.at[idxD.at[b, j]], sem_s))

    for g in range(min(2, outer)):
      a, c = idx_copies(g, g % NB)
      a.start(); c.start()

    def body(g, _):
      b = g % NB
      a, c = idx_copies(g, b)
      a.wait(); c.wait()
      for j in range(KJ):
        a, c = scat_copies(b, j)
        a.start(add=True); c.start(add=True)
      @pl.when(g + 2 < outer)
      def _():
        b2 = (g + 2) % NB
        @pl.when(g >= 2)
        def _():
          for j in range(KJ):
            a, c = scat_copies(b2, j)
            a.wait(); c.wait()
        a, c = idx_copies(g + 2, b2)
        a.start(); c.start()
      return 0
    lax.fori_loop(0, outer, body, 0)

    for g in range(max(0, outer - NB), outer):
      for j in range(KJ):
        a, c = scat_copies(g % NB, j)
        a.wait(); c.wait()

  @pl.when(cid == 0)
  def _():
    pipeline(sid * RC0, O0)
  @pl.when(cid == 1)
  def _():
    pipeline(16 * RC0 + sid * RC1, O1)
  plsc.subcore_barrier()

  _copy_out(acc, out_hbm, cid, sid)


@functools.partial(
    pl.kernel,
    mesh=_mesh,
    compiler_params=_sc_params,
    out_type=jax.ShapeDtypeStruct((NC, NPAD, HID), jnp.float32),
    scratch_types=[
        pltpu.VMEM((NB, KJ, G), jnp.int32),
        pltpu.VMEM((NB, KJ, G), jnp.int32),
        pltpu.VMEM((RB, KJ, G, HID), jnp.float32),
        pltpu.VMEM((128, HID), jnp.float32),
        pltpu.VMEM_SHARED((NPAD, HID), jnp.float32),
        pltpu.SemaphoreType.DMA,
        pltpu.SemaphoreType.DMA,
        pltpu.SemaphoreType.DMA,
    ],
)
def _sc_agg_vec(h_hbm, src_hbm, dst_hbm, out_hbm,
                idxS, idxD, rows, zrows, acc, sem_i, sem_g, sem_s):
  cid = lax.axis_index("c")
  sid = lax.axis_index("s")

  _fill_rows(zrows, 128, jnp.zeros((HID,), jnp.float32))
  _zero_acc(acc, sid, zrows)
  plsc.subcore_barrier()

  def pipeline(r0, outer):
    def idx_copies(g, b):
      r = r0 + g * KJ
      return (pltpu.make_async_copy(src_hbm.at[pl.ds(r, KJ)], idxS.at[b], sem_i),
              pltpu.make_async_copy(dst_hbm.at[pl.ds(r, KJ)], idxD.at[b], sem_i))

    def gath_copy(b, rb, j):
      return pltpu.make_async_copy(h_hbm.at[idxS.at[b, j]], rows.at[rb, j], sem_g)

    def scat_copy(b, rb, j):
      return pltpu.make_async_copy(rows.at[rb, j], acc.at[idxD.at[b, j]], sem_s)

    for g in range(min(2, outer)):
      a, c = idx_copies(g, g % NB)
      a.start(); c.start()

    def body(g, _):
      b = g % NB
      rb = g % RB
      a, c = idx_copies(g, b)
      a.wait(); c.wait()
      @pl.when(g >= 2)
      def _():
        for j in range(KJ):
          scat_copy((g + 2) % NB, rb, j).wait()
      for j in range(KJ):
        gath_copy(b, rb, j).start()
      @pl.when(g + 2 < outer)
      def _():
        a, c = idx_copies(g + 2, (g + 2) % NB)
        a.start(); c.start()
      for j in range(KJ):
        gath_copy(b, rb, j).wait()
      for j in range(KJ):
        scat_copy(b, rb, j).start(add=True)
      return 0
    lax.fori_loop(0, outer, body, 0)

    for g in range(max(0, outer - 2), outer):
      for j in range(KJ):
        scat_copy(g % NB, g % RB, j).wait()

  @pl.when(cid == 0)
  def _():
    pipeline(sid * RC0, O0)
  @pl.when(cid == 1)
  def _():
    pipeline(16 * RC0 + sid * RC1, O1)
  plsc.subcore_barrier()

  _copy_out(acc, out_hbm, cid, sid)


BROW = 800
BP8 = 1280
GP8 = NP8 // BP8


def _tc_norms_body(ds0, ds1, dd0, dd1, x, ns, nd, xs):
  od = jnp.maximum(ds0[...] + ds1[...], 1.0)
  idg = jnp.maximum(dd0[...] + dd1[...], 1.0)
  ns_v = lax.rsqrt(od)
  ns[...] = ns_v
  nd[...] = lax.rsqrt(idg)
  xs[...] = x[...] * ns_v


def _tc_norms(ds0, ds1, dd0, dd1, x):
  spec = pl.BlockSpec((BROW, 128), lambda: (0, 0))
  return pl.pallas_call(
      _tc_norms_body,
      in_specs=[spec] * 5,
      out_specs=[spec] * 3,
      out_shape=[jax.ShapeDtypeStruct((BROW, 128), jnp.float32)] * 3,
  )(ds0, ds1, dd0, dd1, x)


def _p8_spec():
  return pl.BlockSpec((BP8, 128), lambda i: (i, 0))


def _w_spec():
  return pl.BlockSpec((128, 128), lambda i: (0, 0))


def _r_spec():
  return pl.BlockSpec((1, 128), lambda i: (0, 0))


def _tc_h0_body(xsr, w0, out):
  out[...] = xsr[...] * w0[...]


def _tc_h0(xs_rep, w0_big):
  return pl.pallas_call(
      _tc_h0_body,
      grid=(GP8,),
      in_specs=[_p8_spec(), _r_spec()],
      out_specs=_p8_spec(),
      out_shape=jax.ShapeDtypeStruct((NP8, 128), jnp.float32),
  )(xs_rep, w0_big)


def _tc_affine_body(a0, a1, ndr, nsr, b, out):
  t = (a0[...] + a1[...]) * ndr[...] + b[...]
  out[...] = jnp.maximum(t, 0.0) * nsr[...]


def _tc_affine(a0, a1, ndr, nsr, b_big):
  return pl.pallas_call(
      _tc_affine_body,
      grid=(GP8,),
      in_specs=[_p8_spec(), _p8_spec(), _p8_spec(), _p8_spec(), _r_spec()],
      out_specs=_p8_spec(),
      out_shape=jax.ShapeDtypeStruct((NP8, 128), jnp.float32),
  )(a0, a1, ndr, nsr, b_big)


def _make_tc_dense(relu, scale_src):
  def body(a0, a1, ndr, nsr, w, b, out):
    t = (a0[...] + a1[...]) * ndr[...]
    h = lax.dot_general(t, w[...], (((1,), (0,)), ((), ())),
                        preferred_element_type=jnp.float32) + b[...]
    if relu:
      h = jnp.maximum(h, 0.0)
    if scale_src:
      h = h * nsr[...]
    out[...] = h

  def run(a0, a1, ndr, nsr, w_big, b_big):
    return pl.pallas_call(
        body,
        grid=(GP8,),
        in_specs=[_p8_spec(), _p8_spec(), _p8_spec(), _p8_spec(),
                  _w_spec(), _r_spec()],
        out_specs=_p8_spec(),
        out_shape=jax.ShapeDtypeStruct((NP8, 128), jnp.float32),
    )(a0, a1, ndr, nsr, w_big, b_big)
  return run


_tc_dense_mid = _make_tc_dense(relu=True, scale_src=True)
_tc_dense_last = _make_tc_dense(relu=False, scale_src=False)


def _rep16(v_2d):
  return jnp.repeat(v_2d.reshape(NPAD), HID).reshape(NP8, 128)


def _packed(agg_part):
  return agg_part.reshape(NP8, 128)


@jax.jit
def kernel(x, edge_index, W0, b0, W1, b1, W2, b2):
  src = edge_index[0]
  dst = edge_index[1]
  pad = EROWS * G - E
  src_p = jnp.concatenate([src, jnp.full((pad,), NPAD - 1, jnp.int32)]).reshape(EROWS, G)
  dst_p = jnp.concatenate([dst, jnp.full((pad,), NPAD - 1, jnp.int32)]).reshape(EROWS, G)

  deg_p = _sc_degrees(src_p, dst_p)
  ds0 = deg_p[0, :, 0].reshape(BROW, 128)
  ds1 = deg_p[1, :, 0].reshape(BROW, 128)
  dd0 = deg_p[0, :, 1].reshape(BROW, 128)
  dd1 = deg_p[1, :, 1].reshape(BROW, 128)
  x_pad = jnp.concatenate([x.reshape(N), jnp.zeros((NPAD - N,), jnp.float32)])

  ns, nd, xs = _tc_norms(ds0, ds1, dd0, dd1, x_pad.reshape(BROW, 128))
  ns_rep = _rep16(ns)
  nd_rep = _rep16(nd)
  xs_rep = _rep16(xs)

  w0_big = jnp.tile(W0.reshape(HID), 8).reshape(1, 128)
  b0_big = jnp.tile(b0, 8).reshape(1, 128)
  w1_big = jnp.kron(jnp.eye(8, dtype=jnp.float32), W1)
  b1_big = jnp.tile(b1, 8).reshape(1, 128)
  w2_big = jnp.kron(jnp.eye(8, dtype=jnp.float32), W2)
  b2_big = jnp.tile(b2, 8).reshape(1, 128)

  h0 = _tc_h0(xs_rep, w0_big)
  agg1 = _sc_agg_vec(h0.reshape(NPAD, HID), src_p, dst_p)
  h1 = _tc_affine(_packed(agg1[0]), _packed(agg1[1]), nd_rep, ns_rep, b0_big)

  agg2 = _sc_agg_vec(h1.reshape(NPAD, HID), src_p, dst_p)
  h2 = _tc_dense_mid(_packed(agg2[0]), _packed(agg2[1]), nd_rep, ns_rep,
                     w1_big, b1_big)

  agg3 = _sc_agg_vec(h2.reshape(NPAD, HID), src_p, dst_p)
  out = _tc_dense_last(_packed(agg3[0]), _packed(agg3[1]), nd_rep, ns_rep,
                       w2_big, b2_big)
  return out.reshape(NPAD, HID)[:N]


# speedup vs baseline: 1.1603x; 1.1603x over previous
import functools

import jax
import jax.numpy as jnp
from jax import lax
from jax.experimental import pallas as pl
from jax.experimental.pallas import tpu as pltpu
from jax.experimental.pallas import tpu_sc as plsc

N = 100000
E = 3200000
HID = 16

NPAD = 102400
NP8 = NPAD // 8
NREP = NPAD * HID

NC = 2
NS = 16
NW = NC * NS
G = 128
EROWS = 25600
KJ = 4
NB = 4
RB = 2
RC0 = 1104
RC1 = 496
O0 = RC0 // KJ
O1 = RC1 // KJ

_mesh = plsc.VectorSubcoreMesh(core_axis_name="c", subcore_axis_name="s")
_sc_params = pltpu.CompilerParams(use_tc_tiling_on_sc=False)


def _fill_rows(buf, nrows, vec):
  def body(i, _):
    buf[i] = vec
    return 0
  lax.fori_loop(0, nrows, body, 0)


def _zero_acc(acc, sid, zrows):
  chunk = NPAD // NS
  def zb(i, _):
    pltpu.sync_copy(zrows, acc.at[pl.ds(sid * chunk + i * 128, 128)])
    return 0
  lax.fori_loop(0, chunk // 128, zb, 0)


def _copy_out(acc, out_hbm, cid, sid):
  chunk = NPAD // NS
  off = sid * chunk
  pltpu.sync_copy(acc.at[pl.ds(off, chunk)], out_hbm.at[cid, pl.ds(off, chunk)])


@functools.partial(
    pl.kernel,
    mesh=_mesh,
    compiler_params=_sc_params,
    out_type=jax.ShapeDtypeStruct((NC, NPAD, HID), jnp.float32),
    scratch_types=[
        pltpu.VMEM((NB, KJ, G), jnp.int32),
        pltpu.VMEM((NB, KJ, G), jnp.int32),
        pltpu.VMEM((G, HID), jnp.float32),
        pltpu.VMEM((G, HID), jnp.float32),
        pltpu.VMEM((128, HID), jnp.float32),
        pltpu.VMEM_SHARED((NPAD, HID), jnp.float32),
        pltpu.SemaphoreType.DMA,
        pltpu.SemaphoreType.DMA,
    ],
)
def _sc_degrees(src_hbm, dst_hbm, out_hbm, idxS, idxD, e0buf, e1buf, zrows, acc,
                sem_i, sem_s):
  cid = lax.axis_index("c")
  sid = lax.axis_index("s")

  lane = lax.iota(jnp.int32, HID)
  zvec = jnp.zeros((HID,), jnp.float32)
  e0 = jnp.where(lane == 0, 1.0, 0.0).astype(jnp.float32)
  e1 = jnp.where(lane == 1, 1.0, 0.0).astype(jnp.float32)
  _fill_rows(e0buf, G, e0)
  _fill_rows(e1buf, G, e1)
  _fill_rows(zrows, 128, zvec)

  _zero_acc(acc, sid, zrows)
  plsc.subcore_barrier()

  def pipeline(r0, outer):
    def idx_copies(g, b):
      r = r0 + g * KJ
      return (pltpu.make_async_copy(src_hbm.at[pl.ds(r, KJ)], idxS.at[b], sem_i),
              pltpu.make_async_copy(dst_hbm.at[pl.ds(r, KJ)], idxD.at[b], sem_i))

    def scat_copies(b, j):
      return (pltpu.make_async_copy(e0buf, acc.at[idxS.at[b, j]], sem_s),
              pltpu.make_async_copy(e1buf, acc.at[idxD.at[b, j]], sem_s))

    for g in range(min(2, outer)):
      a, c = idx_copies(g, g % NB)
      a.start(); c.start()

    def body(g, _):
      b = g % NB
      a, c = idx_copies(g, b)
      a.wait(); c.wait()
      for j in range(KJ):
        a, c = scat_copies(b, j)
        a.start(add=True); c.start(add=True)
      @pl.when(g + 2 < outer)
      def _():
        b2 = (g + 2) % NB
        @pl.when(g >= 2)
        def _():
          for j in range(KJ):
            a, c = scat_copies(b2, j)
            a.wait(); c.wait()
        a, c = idx_copies(g + 2, b2)
        a.start(); c.start()
      return 0
    lax.fori_loop(0, outer, body, 0)

    for g in range(max(0, outer - NB), outer):
      for j in range(KJ):
        a, c = scat_copies(g % NB, j)
        a.wait(); c.wait()

  @pl.when(cid == 0)
  def _():
    pipeline(sid * RC0, O0)
  @pl.when(cid == 1)
  def _():
    pipeline(16 * RC0 + sid * RC1, O1)
  plsc.subcore_barrier()

  _copy_out(acc, out_hbm, cid, sid)


@functools.partial(
    pl.kernel,
    mesh=_mesh,
    compiler_params=_sc_params,
    out_type=jax.ShapeDtypeStruct((NC, NPAD, HID), jnp.float32),
    scratch_types=[
        pltpu.VMEM((NB, KJ, G), jnp.int32),
        pltpu.VMEM((NB, KJ, G), jnp.int32),
        pltpu.VMEM((RB, KJ, G, HID), jnp.float32),
        pltpu.VMEM((128, HID), jnp.float32),
        pltpu.VMEM_SHARED((NPAD, HID), jnp.float32),
        pltpu.SemaphoreType.DMA,
        pltpu.SemaphoreType.DMA,
        pltpu.SemaphoreType.DMA,
    ],
)
def _sc_agg_vec(h_hbm, src_hbm, dst_hbm, out_hbm,
                idxS, idxD, rows, zrows, acc, sem_i, sem_g, sem_s):
  cid = lax.axis_index("c")
  sid = lax.axis_index("s")

  _fill_rows(zrows, 128, jnp.zeros((HID,), jnp.float32))
  _zero_acc(acc, sid, zrows)
  plsc.subcore_barrier()

  def pipeline(r0, outer):
    def idx_copies(g, b):
      r = r0 + g * KJ
      return (pltpu.make_async_copy(src_hbm.at[pl.ds(r, KJ)], idxS.at[b], sem_i),
              pltpu.make_async_copy(dst_hbm.at[pl.ds(r, KJ)], idxD.at[b], sem_i))

    def gath_copy(b, rb, j):
      return pltpu.make_async_copy(h_hbm.at[idxS.at[b, j]], rows.at[rb, j], sem_g)

    def scat_copy(b, rb, j):
      return pltpu.make_async_copy(rows.at[rb, j], acc.at[idxD.at[b, j]], sem_s)

    for g in range(min(2, outer)):
      a, c = idx_copies(g, g % NB)
      a.start(); c.start()

    def body(g, _):
      b = g % NB
      rb = g % RB
      a, c = idx_copies(g, b)
      a.wait(); c.wait()
      @pl.when(g >= 2)
      def _():
        for j in range(KJ):
          scat_copy((g + 2) % NB, rb, j).wait()
      for j in range(KJ):
        gath_copy(b, rb, j).start()
      @pl.when(g + 2 < outer)
      def _():
        a, c = idx_copies(g + 2, (g + 2) % NB)
        a.start(); c.start()
      for j in range(KJ):
        gath_copy(b, rb, j).wait()
      for j in range(KJ):
        scat_copy(b, rb, j).start(add=True)
      return 0
    lax.fori_loop(0, outer, body, 0)

    for g in range(max(0, outer - 2), outer):
      for j in range(KJ):
        scat_copy(g % NB, g % RB, j).wait()

  @pl.when(cid == 0)
  def _():
    pipeline(sid * RC0, O0)
  @pl.when(cid == 1)
  def _():
    pipeline(16 * RC0 + sid * RC1, O1)
  plsc.subcore_barrier()

  _copy_out(acc, out_hbm, cid, sid)


BROW = 800
BP8 = 1280
GP8 = NP8 // BP8


def _tc_norms_body(ds0, ds1, dd0, dd1, x, ns, nd, xs):
  od = jnp.maximum(ds0[...] + ds1[...], 1.0)
  idg = jnp.maximum(dd0[...] + dd1[...], 1.0)
  ns_v = lax.rsqrt(od)
  ns[...] = ns_v
  nd[...] = lax.rsqrt(idg)
  xs[...] = x[...] * ns_v


def _tc_norms(ds0, ds1, dd0, dd1, x):
  spec = pl.BlockSpec((BROW, 128), lambda: (0, 0))
  return pl.pallas_call(
      _tc_norms_body,
      in_specs=[spec] * 5,
      out_specs=[spec] * 3,
      out_shape=[jax.ShapeDtypeStruct((BROW, 128), jnp.float32)] * 3,
  )(ds0, ds1, dd0, dd1, x)


def _p8_spec():
  return pl.BlockSpec((BP8, 128), lambda i: (i, 0))


def _w_spec():
  return pl.BlockSpec((128, 128), lambda i: (0, 0))


def _r_spec():
  return pl.BlockSpec((1, 128), lambda i: (0, 0))


def _tc_h0_body(xsr, w0, out):
  out[...] = xsr[...] * w0[...]


def _tc_h0(xs_rep, w0_big):
  return pl.pallas_call(
      _tc_h0_body,
      grid=(GP8,),
      in_specs=[_p8_spec(), _r_spec()],
      out_specs=_p8_spec(),
      out_shape=jax.ShapeDtypeStruct((NP8, 128), jnp.float32),
  )(xs_rep, w0_big)


def _tc_affine_body(a0, a1, ndr, nsr, b, out):
  t = (a0[...] + a1[...]) * ndr[...] + b[...]
  out[...] = jnp.maximum(t, 0.0) * nsr[...]


def _tc_affine(a0, a1, ndr, nsr, b_big):
  return pl.pallas_call(
      _tc_affine_body,
      grid=(GP8,),
      in_specs=[_p8_spec(), _p8_spec(), _p8_spec(), _p8_spec(), _r_spec()],
      out_specs=_p8_spec(),
      out_shape=jax.ShapeDtypeStruct((NP8, 128), jnp.float32),
  )(a0, a1, ndr, nsr, b_big)


def _make_tc_dense(relu, scale_src):
  def body(a0, a1, ndr, nsr, w, b, out):
    t = (a0[...] + a1[...]) * ndr[...]
    h = lax.dot_general(t, w[...], (((1,), (0,)), ((), ())),
                        preferred_element_type=jnp.float32) + b[...]
    if relu:
      h = jnp.maximum(h, 0.0)
    if scale_src:
      h = h * nsr[...]
    out[...] = h

  def run(a0, a1, ndr, nsr, w_big, b_big):
    return pl.pallas_call(
        body,
        grid=(GP8,),
        in_specs=[_p8_spec(), _p8_spec(), _p8_spec(), _p8_spec(),
                  _w_spec(), _r_spec()],
        out_specs=_p8_spec(),
        out_shape=jax.ShapeDtypeStruct((NP8, 128), jnp.float32),
    )(a0, a1, ndr, nsr, w_big, b_big)
  return run


_tc_dense_mid = _make_tc_dense(relu=True, scale_src=True)
_tc_dense_last = _make_tc_dense(relu=False, scale_src=False)


def _rep16(v_2d):
  return jnp.repeat(v_2d.reshape(NPAD), HID).reshape(NP8, 128)


def _packed(agg_part):
  return agg_part.reshape(NP8, 128)


@jax.jit
def kernel(x, edge_index, W0, b0, W1, b1, W2, b2):
  src = edge_index[0]
  dst = edge_index[1]
  pad = EROWS * G - E
  src_p = jnp.concatenate([src, jnp.full((pad,), NPAD - 1, jnp.int32)]).reshape(EROWS, G)
  dst_p = jnp.concatenate([dst, jnp.full((pad,), NPAD - 1, jnp.int32)]).reshape(EROWS, G)

  deg_p = _sc_degrees(src_p, dst_p)
  ds0 = deg_p[0, :, 0].reshape(BROW, 128)
  ds1 = deg_p[1, :, 0].reshape(BROW, 128)
  dd0 = deg_p[0, :, 1].reshape(BROW, 128)
  dd1 = deg_p[1, :, 1].reshape(BROW, 128)
  x_pad = jnp.concatenate([x.reshape(N), jnp.zeros((NPAD - N,), jnp.float32)])

  ns, nd, xs = _tc_norms(ds0, ds1, dd0, dd1, x_pad.reshape(BROW, 128))
  ns_rep = _rep16(ns)
  nd_rep = _rep16(nd)
  xs_rep = _rep16(xs)

  w0_big = jnp.tile(W0.reshape(HID), 8).reshape(1, 128)
  b0_big = jnp.tile(b0, 8).reshape(1, 128)
  w1_big = jnp.kron(jnp.eye(8, dtype=jnp.float32), W1)
  b1_big = jnp.tile(b1, 8).reshape(1, 128)
  w2_big = jnp.kron(jnp.eye(8, dtype=jnp.float32), W2)
  b2_big = jnp.tile(b2, 8).reshape(1, 128)

  h0 = _tc_h0(xs_rep, w0_big)
  agg1 = _sc_agg_vec(h0.reshape(NPAD, HID), src_p, dst_p)
  h1 = _tc_affine(_packed(agg1[0]), _packed(agg1[1]), nd_rep, ns_rep, b0_big)

  agg2 = _sc_agg_vec(h1.reshape(NPAD, HID), src_p, dst_p)
  h2 = _tc_dense_mid(_packed(agg2[0]), _packed(agg2[1]), nd_rep, ns_rep,
                     w1_big, b1_big)

  agg3 = _sc_agg_vec(h2.reshape(NPAD, HID), src_p, dst_p)
  out = _tc_dense_last(_packed(agg3[0]), _packed(agg3[1]), nd_rep, ns_rep,
                       w2_big, b2_big)
  return out.reshape(NPAD, HID)[:N]


# speedup vs baseline: 1.4707x; 1.2676x over previous
import functools

import jax
import jax.numpy as jnp
from jax import lax
from jax.experimental import pallas as pl
from jax.experimental.pallas import tpu as pltpu
from jax.experimental.pallas import tpu_sc as plsc

N = 100000
E = 3200000
HID = 16

NPAD = 102400
NP8 = NPAD // 8
NREP = NPAD * HID

NC = 2
NS = 16
NW = NC * NS
G = 128
EROWS = 25600
KJ = 4
NB = 4
RB = 2
RA0 = 960
RA1 = 640
RD0 = 1040
RD1 = 560

_mesh = plsc.VectorSubcoreMesh(core_axis_name="c", subcore_axis_name="s")
_sc_params = pltpu.CompilerParams(use_tc_tiling_on_sc=False)


def _fill_rows(buf, nrows, vec):
  def body(i, _):
    buf[i] = vec
    return 0
  lax.fori_loop(0, nrows, body, 0)


def _zero_acc(acc, sid, zrows):
  chunk = NPAD // NS
  def zb(i, _):
    pltpu.sync_copy(zrows, acc.at[pl.ds(sid * chunk + i * 128, 128)])
    return 0
  lax.fori_loop(0, chunk // 128, zb, 0)


def _copy_out(acc, out_hbm, cid, sid):
  chunk = NPAD // NS
  off = sid * chunk
  pltpu.sync_copy(acc.at[pl.ds(off, chunk)], out_hbm.at[cid, pl.ds(off, chunk)])


@functools.partial(
    pl.kernel,
    mesh=_mesh,
    compiler_params=_sc_params,
    out_type=jax.ShapeDtypeStruct((NC, NPAD, HID), jnp.float32),
    scratch_types=[
        pltpu.VMEM((NB, KJ, G), jnp.int32),
        pltpu.VMEM((NB, KJ, G), jnp.int32),
        pltpu.VMEM((G, HID), jnp.float32),
        pltpu.VMEM((G, HID), jnp.float32),
        pltpu.VMEM((128, HID), jnp.float32),
        pltpu.VMEM_SHARED((NPAD, HID), jnp.float32),
        pltpu.SemaphoreType.DMA,
        pltpu.SemaphoreType.DMA,
    ],
)
def _sc_degrees(src_hbm, dst_hbm, out_hbm, idxS, idxD, e0buf, e1buf, zrows, acc,
                sem_i, sem_s):
  cid = lax.axis_index("c")
  sid = lax.axis_index("s")

  lane = lax.iota(jnp.int32, HID)
  zvec = jnp.zeros((HID,), jnp.float32)
  e0 = jnp.where(lane == 0, 1.0, 0.0).astype(jnp.float32)
  e1 = jnp.where(lane == 1, 1.0, 0.0).astype(jnp.float32)
  _fill_rows(e0buf, G, e0)
  _fill_rows(e1buf, G, e1)
  _fill_rows(zrows, 128, zvec)

  _zero_acc(acc, sid, zrows)
  plsc.subcore_barrier()

  def pipeline(r0, outer):
    def idx_copies(g, b):
      r = r0 + g * KJ
      return (pltpu.make_async_copy(src_hbm.at[pl.ds(r, KJ)], idxS.at[b], sem_i),
              pltpu.make_async_copy(dst_hbm.at[pl.ds(r, KJ)], idxD.at[b], sem_i))

    def scat_copies(b, j):
      return (pltpu.make_async_copy(e0buf, acc.at[idxS.at[b, j]], sem_s),
              pltpu.make_async_copy(e1buf, acc.at[idxD.at[b, j]], sem_s))

    for g in range(min(2, outer)):
      a, c = idx_copies(g, g % NB)
      a.start(); c.start()

    def body(g, _):
      b = g % NB
      a, c = idx_copies(g, b)
      a.wait(); c.wait()
      for j in range(KJ):
        a, c = scat_copies(b, j)
        a.start(add=True); c.start(add=True)
      @pl.when(g + 2 < outer)
      def _():
        b2 = (g + 2) % NB
        @pl.when(g >= 2)
        def _():
          for j in range(KJ):
            a, c = scat_copies(b2, j)
            a.wait(); c.wait()
        a, c = idx_copies(g + 2, b2)
        a.start(); c.start()
      return 0
    lax.fori_loop(0, outer, body, 0)

    for g in range(max(0, outer - NB), outer):
      for j in range(KJ):
        a, c = scat_copies(g % NB, j)
        a.wait(); c.wait()

  @pl.when(cid == 0)
  def _():
    pipeline(sid * RD0, RD0 // KJ)
  @pl.when(cid == 1)
  def _():
    pipeline(16 * RD0 + sid * RD1, RD1 // KJ)
  plsc.subcore_barrier()

  _copy_out(acc, out_hbm, cid, sid)


@functools.partial(
    pl.kernel,
    mesh=_mesh,
    compiler_params=_sc_params,
    out_type=jax.ShapeDtypeStruct((NC, NPAD, HID), jnp.float32),
    scratch_types=[
        pltpu.VMEM((NB, KJ, G), jnp.int32),
        pltpu.VMEM((NB, KJ, G), jnp.int32),
        pltpu.VMEM((RB, KJ, G, HID), jnp.float32),
        pltpu.VMEM((128, HID), jnp.float32),
        pltpu.VMEM_SHARED((NPAD, HID), jnp.float32),
        pltpu.SemaphoreType.DMA,
        pltpu.SemaphoreType.DMA,
        pltpu.SemaphoreType.DMA,
    ],
)
def _sc_agg_vec(h_hbm, src_hbm, dst_hbm, out_hbm,
                idxS, idxD, rows, zrows, acc, sem_i, sem_g, sem_s):
  cid = lax.axis_index("c")
  sid = lax.axis_index("s")

  _fill_rows(zrows, 128, jnp.zeros((HID,), jnp.float32))
  _zero_acc(acc, sid, zrows)
  plsc.subcore_barrier()

  def pipeline(r0, outer):
    def idx_copies(g, b):
      r = r0 + g * KJ
      return (pltpu.make_async_copy(src_hbm.at[pl.ds(r, KJ)], idxS.at[b], sem_i),
              pltpu.make_async_copy(dst_hbm.at[pl.ds(r, KJ)], idxD.at[b], sem_i))

    def gath_copy(b, rb, j):
      return pltpu.make_async_copy(h_hbm.at[idxS.at[b, j]], rows.at[rb, j], sem_g)

    def scat_copy(b, rb, j):
      return pltpu.make_async_copy(rows.at[rb, j], acc.at[idxD.at[b, j]], sem_s)

    for g in range(min(2, outer)):
      a, c = idx_copies(g, g % NB)
      a.start(); c.start()

    def body(g, _):
      b = g % NB
      rb = g % RB
      a, c = idx_copies(g, b)
      a.wait(); c.wait()
      @pl.when(g >= 2)
      def _():
        for j in range(KJ):
          scat_copy((g + 2) % NB, rb, j).wait()
      for j in range(KJ):
        gath_copy(b, rb, j).start()
      @pl.when(g + 2 < outer)
      def _():
        a, c = idx_copies(g + 2, (g + 2) % NB)
        a.start(); c.start()
      for j in range(KJ):
        gath_copy(b, rb, j).wait()
      for j in range(KJ):
        scat_copy(b, rb, j).start(add=True)
      return 0
    lax.fori_loop(0, outer, body, 0)

    for g in range(max(0, outer - 2), outer):
      for j in range(KJ):
        scat_copy(g % NB, g % RB, j).wait()

  @pl.when(cid == 0)
  def _():
    pipeline(sid * RA0, RA0 // KJ)
  @pl.when(cid == 1)
  def _():
    pipeline(16 * RA0 + sid * RA1, RA1 // KJ)
  plsc.subcore_barrier()

  _copy_out(acc, out_hbm, cid, sid)


BP8 = 1280
GP8 = NP8 // BP8


def _p8_spec():
  return pl.BlockSpec((BP8, 128), lambda i: (i, 0))


def _p8hi_spec():
  return pl.BlockSpec((BP8, 128), lambda i: (i + GP8, 0))


def _w_spec():
  return pl.BlockSpec((128, 128), lambda i: (0, 0))


def _r_spec():
  return pl.BlockSpec((1, 128), lambda i: (0, 0))


def _tc_norms_body(d0, d1, xr, s0, s1, w0, nsr, ndr, h0):
  d = d0[...] + d1[...]
  od = lax.dot_general(d, s0[...], (((1,), (0,)), ((), ())),
                       preferred_element_type=jnp.float32)
  idg = lax.dot_general(d, s1[...], (((1,), (0,)), ((), ())),
                        preferred_element_type=jnp.float32)
  ns_v = lax.rsqrt(jnp.maximum(od, 1.0))
  nsr[...] = ns_v
  ndr[...] = lax.rsqrt(jnp.maximum(idg, 1.0))
  h0[...] = xr[...] * ns_v * w0[...]


def _tc_norms(degv, x_rep, s0, s1, w0_big):
  return pl.pallas_call(
      _tc_norms_body,
      grid=(GP8,),
      in_specs=[_p8_spec(), _p8hi_spec(), _p8_spec(), _w_spec(), _w_spec(),
                _r_spec()],
      out_specs=[_p8_spec()] * 3,
      out_shape=[jax.ShapeDtypeStruct((NP8, 128), jnp.float32)] * 3,
  )(degv, degv, x_rep, s0, s1, w0_big)


def _tc_affine_body(a0, a1, ndr, nsr, b, out):
  t = (a0[...] + a1[...]) * ndr[...] + b[...]
  out[...] = jnp.maximum(t, 0.0) * nsr[...]


def _tc_affine(aggv, ndr, nsr, b_big):
  return pl.pallas_call(
      _tc_affine_body,
      grid=(GP8,),
      in_specs=[_p8_spec(), _p8hi_spec(), _p8_spec(), _p8_spec(), _r_spec()],
      out_specs=_p8_spec(),
      out_shape=jax.ShapeDtypeStruct((NP8, 128), jnp.float32),
  )(aggv, aggv, ndr, nsr, b_big)


def _make_tc_dense(relu, scale_src):
  def body(a0, a1, ndr, nsr, w, b, out):
    t = (a0[...] + a1[...]) * ndr[...]
    h = lax.dot_general(t, w[...], (((1,), (0,)), ((), ())),
                        preferred_element_type=jnp.float32) + b[...]
    if relu:
      h = jnp.maximum(h, 0.0)
    if scale_src:
      h = h * nsr[...]
    out[...] = h

  def run(aggv, ndr, nsr, w_big, b_big):
    return pl.pallas_call(
        body,
        grid=(GP8,),
        in_specs=[_p8_spec(), _p8hi_spec(), _p8_spec(), _p8_spec(),
                  _w_spec(), _r_spec()],
        out_specs=_p8_spec(),
        out_shape=jax.ShapeDtypeStruct((NP8, 128), jnp.float32),
    )(aggv, aggv, ndr, nsr, w_big, b_big)
  return run


_tc_dense_mid = _make_tc_dense(relu=True, scale_src=True)
_tc_dense_last = _make_tc_dense(relu=False, scale_src=False)


def _flat(agg):
  return agg.reshape(2 * NP8, 128)


@jax.jit
def kernel(x, edge_index, W0, b0, W1, b1, W2, b2):
  src = edge_index[0]
  dst = edge_index[1]
  pad = EROWS * G - E
  src_p = jnp.concatenate([src, jnp.full((pad,), NPAD - 1, jnp.int32)]).reshape(EROWS, G)
  dst_p = jnp.concatenate([dst, jnp.full((pad,), NPAD - 1, jnp.int32)]).reshape(EROWS, G)

  x_pad = jnp.concatenate([x.reshape(N), jnp.zeros((NPAD - N,), jnp.float32)])
  x_rep = jnp.repeat(x_pad, HID).reshape(NP8, 128)
  ii = jnp.arange(128)
  base = (ii[None, :] // HID) * HID
  s0 = (ii[:, None] == base).astype(jnp.float32)
  s1 = (ii[:, None] == base + 1).astype(jnp.float32)

  w0_big = jnp.tile(W0.reshape(HID), 8).reshape(1, 128)
  b0_big = jnp.tile(b0, 8).reshape(1, 128)
  w1_big = jnp.kron(jnp.eye(8, dtype=jnp.float32), W1)
  b1_big = jnp.tile(b1, 8).reshape(1, 128)
  w2_big = jnp.kron(jnp.eye(8, dtype=jnp.float32), W2)
  b2_big = jnp.tile(b2, 8).reshape(1, 128)

  deg_p = _sc_degrees(src_p, dst_p)
  ns_rep, nd_rep, h0 = _tc_norms(_flat(deg_p), x_rep, s0, s1, w0_big)

  agg1 = _sc_agg_vec(h0.reshape(NPAD, HID), src_p, dst_p)
  h1 = _tc_affine(_flat(agg1), nd_rep, ns_rep, b0_big)

  agg2 = _sc_agg_vec(h1.reshape(NPAD, HID), src_p, dst_p)
  h2 = _tc_dense_mid(_flat(agg2), nd_rep, ns_rep, w1_big, b1_big)

  agg3 = _sc_agg_vec(h2.reshape(NPAD, HID), src_p, dst_p)
  out = _tc_dense_last(_flat(agg3), nd_rep, ns_rep, w2_big, b2_big)
  return out.reshape(NPAD, HID)[:N]


# speedup vs baseline: 1.5515x; 1.0549x over previous
import functools

import jax
import jax.numpy as jnp
from jax import lax
from jax.experimental import pallas as pl
from jax.experimental.pallas import tpu as pltpu
from jax.experimental.pallas import tpu_sc as plsc

N = 100000
E = 3200000
HID = 16

NPAD = 102400
NP8 = NPAD // 8
NREP = NPAD * HID

NC = 2
NS = 16
NW = NC * NS
G = 128
EROWS = 25600
KJ = 4
NB = 4
RB = 2
RA0 = 1104
RA1 = 496
RD0 = 1104
RD1 = 496

_mesh = plsc.VectorSubcoreMesh(core_axis_name="c", subcore_axis_name="s")
_sc_params = pltpu.CompilerParams(use_tc_tiling_on_sc=False)


def _fill_rows(buf, nrows, vec):
  def body(i, _):
    buf[i] = vec
    return 0
  lax.fori_loop(0, nrows, body, 0)


def _zero_acc(acc, sid, zrows):
  chunk = NPAD // NS
  def zb(i, _):
    pltpu.sync_copy(zrows, acc.at[pl.ds(sid * chunk + i * 128, 128)])
    return 0
  lax.fori_loop(0, chunk // 128, zb, 0)


def _copy_out(acc, out_hbm, cid, sid):
  chunk = NPAD // NS
  off = sid * chunk
  pltpu.sync_copy(acc.at[pl.ds(off, chunk)], out_hbm.at[cid, pl.ds(off, chunk)])


@functools.partial(
    pl.kernel,
    mesh=_mesh,
    compiler_params=_sc_params,
    out_type=jax.ShapeDtypeStruct((NC, NPAD, HID), jnp.float32),
    scratch_types=[
        pltpu.VMEM((NB, KJ, G), jnp.int32),
        pltpu.VMEM((NB, KJ, G), jnp.int32),
        pltpu.VMEM((G, HID), jnp.float32),
        pltpu.VMEM((G, HID), jnp.float32),
        pltpu.VMEM((128, HID), jnp.float32),
        pltpu.VMEM_SHARED((NPAD, HID), jnp.float32),
        pltpu.SemaphoreType.DMA,
        pltpu.SemaphoreType.DMA,
    ],
)
def _sc_degrees(src_hbm, dst_hbm, out_hbm, idxS, idxD, e0buf, e1buf, zrows, acc,
                sem_i, sem_s):
  cid = lax.axis_index("c")
  sid = lax.axis_index("s")

  lane = lax.iota(jnp.int32, HID)
  zvec = jnp.zeros((HID,), jnp.float32)
  e0 = jnp.where(lane == 0, 1.0, 0.0).astype(jnp.float32)
  e1 = jnp.where(lane == 1, 1.0, 0.0).astype(jnp.float32)
  _fill_rows(e0buf, G, e0)
  _fill_rows(e1buf, G, e1)
  _fill_rows(zrows, 128, zvec)

  _zero_acc(acc, sid, zrows)
  plsc.subcore_barrier()

  def pipeline(r0, outer):
    def idx_copies(g, b):
      r = r0 + g * KJ
      return (pltpu.make_async_copy(src_hbm.at[pl.ds(r, KJ)], idxS.at[b], sem_i),
              pltpu.make_async_copy(dst_hbm.at[pl.ds(r, KJ)], idxD.at[b], sem_i))

    def scat_copies(b, j):
      return (pltpu.make_async_copy(e0buf, acc.at[idxS.at[b, j]], sem_s),
              pltpu.make_async_copy(e1buf, acc.at[idxD.at[b, j]], sem_s))

    for g in range(min(2, outer)):
      a, c = idx_copies(g, g % NB)
      a.start(); c.start()

    def body(g, _):
      b = g % NB
      a, c = idx_copies(g, b)
      a.wait(); c.wait()
      for j in range(KJ):
        a, c = scat_copies(b, j)
        a.start(add=True); c.start(add=True)
      @pl.when(g + 2 < outer)
      def _():
        b2 = (g + 2) % NB
        @pl.when(g >= 2)
        def _():
          for j in range(KJ):
            a, c = scat_copies(b2, j)
            a.wait(); c.wait()
        a, c = idx_copies(g + 2, b2)
        a.start(); c.start()
      return 0
    lax.fori_loop(0, outer, body, 0)

    for g in range(max(0, outer - NB), outer):
      for j in range(KJ):
        a, c = scat_copies(g % NB, j)
        a.wait(); c.wait()

  @pl.when(cid == 0)
  def _():
    pipeline(sid * RD0, RD0 // KJ)
  @pl.when(cid == 1)
  def _():
    pipeline(16 * RD0 + sid * RD1, RD1 // KJ)
  plsc.subcore_barrier()

  _copy_out(acc, out_hbm, cid, sid)


@functools.partial(
    pl.kernel,
    mesh=_mesh,
    compiler_params=_sc_params,
    out_type=jax.ShapeDtypeStruct((NC, NPAD, HID), jnp.float32),
    scratch_types=[
        pltpu.VMEM((NB, KJ, G), jnp.int32),
        pltpu.VMEM((NB, KJ, G), jnp.int32),
        pltpu.VMEM((RB, KJ, G, HID), jnp.float32),
        pltpu.VMEM((128, HID), jnp.float32),
        pltpu.VMEM_SHARED((NPAD, HID), jnp.float32),
        pltpu.SemaphoreType.DMA,
        pltpu.SemaphoreType.DMA,
        pltpu.SemaphoreType.DMA,
    ],
)
def _sc_agg_vec(h_hbm, src_hbm, dst_hbm, out_hbm,
                idxS, idxD, rows, zrows, acc, sem_i, sem_g, sem_s):
  cid = lax.axis_index("c")
  sid = lax.axis_index("s")

  _fill_rows(zrows, 128, jnp.zeros((HID,), jnp.float32))
  _zero_acc(acc, sid, zrows)
  plsc.subcore_barrier()

  def pipeline(r0, outer):
    def idx_copies(g, b):
      r = r0 + g * KJ
      return (pltpu.make_async_copy(src_hbm.at[pl.ds(r, KJ)], idxS.at[b], sem_i),
              pltpu.make_async_copy(dst_hbm.at[pl.ds(r, KJ)], idxD.at[b], sem_i))

    def gath_copy(b, rb, j):
      return pltpu.make_async_copy(h_hbm.at[idxS.at[b, j]], rows.at[rb, j], sem_g)

    def scat_copy(b, rb, j):
      return pltpu.make_async_copy(rows.at[rb, j], acc.at[idxD.at[b, j]], sem_s)

    for g in range(min(2, outer)):
      a, c = idx_copies(g, g % NB)
      a.start(); c.start()

    def body(g, _):
      b = g % NB
      rb = g % RB
      a, c = idx_copies(g, b)
      a.wait(); c.wait()
      @pl.when(g >= 2)
      def _():
        for j in range(KJ):
          scat_copy((g + 2) % NB, rb, j).wait()
      for j in range(KJ):
        gath_copy(b, rb, j).start()
      @pl.when(g + 2 < outer)
      def _():
        a, c = idx_copies(g + 2, (g + 2) % NB)
        a.start(); c.start()
      for j in range(KJ):
        gath_copy(b, rb, j).wait()
      for j in range(KJ):
        scat_copy(b, rb, j).start(add=True)
      return 0
    lax.fori_loop(0, outer, body, 0)

    for g in range(max(0, outer - 2), outer):
      for j in range(KJ):
        scat_copy(g % NB, g % RB, j).wait()

  @pl.when(cid == 0)
  def _():
    pipeline(sid * RA0, RA0 // KJ)
  @pl.when(cid == 1)
  def _():
    pipeline(16 * RA0 + sid * RA1, RA1 // KJ)
  plsc.subcore_barrier()

  _copy_out(acc, out_hbm, cid, sid)


BP8 = 1280
GP8 = NP8 // BP8


def _p8_spec():
  return pl.BlockSpec((BP8, 128), lambda i: (i, 0))


def _p8hi_spec():
  return pl.BlockSpec((BP8, 128), lambda i: (i + GP8, 0))


def _w_spec():
  return pl.BlockSpec((128, 128), lambda i: (0, 0))


def _r_spec():
  return pl.BlockSpec((1, 128), lambda i: (0, 0))


def _tc_norms_body(d0, d1, xr, s0, s1, w0, nsr, ndr, h0):
  d = d0[...] + d1[...]
  od = lax.dot_general(d, s0[...], (((1,), (0,)), ((), ())),
                       preferred_element_type=jnp.float32)
  idg = lax.dot_general(d, s1[...], (((1,), (0,)), ((), ())),
                        preferred_element_type=jnp.float32)
  ns_v = lax.rsqrt(jnp.maximum(od, 1.0))
  nsr[...] = ns_v
  ndr[...] = lax.rsqrt(jnp.maximum(idg, 1.0))
  h0[...] = xr[...] * ns_v * w0[...]


def _tc_norms(degv, x_rep, s0, s1, w0_big):
  return pl.pallas_call(
      _tc_norms_body,
      grid=(GP8,),
      in_specs=[_p8_spec(), _p8hi_spec(), _p8_spec(), _w_spec(), _w_spec(),
                _r_spec()],
      out_specs=[_p8_spec()] * 3,
      out_shape=[jax.ShapeDtypeStruct((NP8, 128), jnp.float32)] * 3,
  )(degv, degv, x_rep, s0, s1, w0_big)


def _tc_affine_body(a0, a1, ndr, nsr, b, out):
  t = (a0[...] + a1[...]) * ndr[...] + b[...]
  out[...] = jnp.maximum(t, 0.0) * nsr[...]


def _tc_affine(aggv, ndr, nsr, b_big):
  return pl.pallas_call(
      _tc_affine_body,
      grid=(GP8,),
      in_specs=[_p8_spec(), _p8hi_spec(), _p8_spec(), _p8_spec(), _r_spec()],
      out_specs=_p8_spec(),
      out_shape=jax.ShapeDtypeStruct((NP8, 128), jnp.float32),
  )(aggv, aggv, ndr, nsr, b_big)


def _make_tc_dense(relu, scale_src):
  def body(a0, a1, ndr, nsr, w, b, out):
    t = (a0[...] + a1[...]) * ndr[...]
    h = lax.dot_general(t, w[...], (((1,), (0,)), ((), ())),
                        preferred_element_type=jnp.float32) + b[...]
    if relu:
      h = jnp.maximum(h, 0.0)
    if scale_src:
      h = h * nsr[...]
    out[...] = h

  def run(aggv, ndr, nsr, w_big, b_big):
    return pl.pallas_call(
        body,
        grid=(GP8,),
        in_specs=[_p8_spec(), _p8hi_spec(), _p8_spec(), _p8_spec(),
                  _w_spec(), _r_spec()],
        out_specs=_p8_spec(),
        out_shape=jax.ShapeDtypeStruct((NP8, 128), jnp.float32),
    )(aggv, aggv, ndr, nsr, w_big, b_big)
  return run


_tc_dense_mid = _make_tc_dense(relu=True, scale_src=True)
_tc_dense_last = _make_tc_dense(relu=False, scale_src=False)


def _flat(agg):
  return agg.reshape(2 * NP8, 128)


@jax.jit
def kernel(x, edge_index, W0, b0, W1, b1, W2, b2):
  src = edge_index[0]
  dst = edge_index[1]
  pad = EROWS * G - E
  src_p = jnp.concatenate([src, jnp.full((pad,), NPAD - 1, jnp.int32)]).reshape(EROWS, G)
  dst_p = jnp.concatenate([dst, jnp.full((pad,), NPAD - 1, jnp.int32)]).reshape(EROWS, G)

  x_pad = jnp.concatenate([x.reshape(N), jnp.zeros((NPAD - N,), jnp.float32)])
  x_rep = jnp.repeat(x_pad, HID).reshape(NP8, 128)
  ii = jnp.arange(128)
  base = (ii[None, :] // HID) * HID
  s0 = (ii[:, None] == base).astype(jnp.float32)
  s1 = (ii[:, None] == base + 1).astype(jnp.float32)

  w0_big = jnp.tile(W0.reshape(HID), 8).reshape(1, 128)
  b0_big = jnp.tile(b0, 8).reshape(1, 128)
  w1_big = jnp.kron(jnp.eye(8, dtype=jnp.float32), W1)
  b1_big = jnp.tile(b1, 8).reshape(1, 128)
  w2_big = jnp.kron(jnp.eye(8, dtype=jnp.float32), W2)
  b2_big = jnp.tile(b2, 8).reshape(1, 128)

  deg_p = _sc_degrees(src_p, dst_p)
  ns_rep, nd_rep, h0 = _tc_norms(_flat(deg_p), x_rep, s0, s1, w0_big)

  agg1 = _sc_agg_vec(h0.reshape(NPAD, HID), src_p, dst_p)
  h1 = _tc_affine(_flat(agg1), nd_rep, ns_rep, b0_big)

  agg2 = _sc_agg_vec(h1.reshape(NPAD, HID), src_p, dst_p)
  h2 = _tc_dense_mid(_flat(agg2), nd_rep, ns_rep, w1_big, b1_big)

  agg3 = _sc_agg_vec(h2.reshape(NPAD, HID), src_p, dst_p)
  out = _tc_dense_last(_flat(agg3), nd_rep, ns_rep, w2_big, b2_big)
  return out.reshape(NPAD, HID)[:N]


# speedup vs baseline: 1.5597x; 1.0053x over previous
import functools

import jax
import jax.numpy as jnp
from jax import lax
from jax.experimental import pallas as pl
from jax.experimental.pallas import tpu as pltpu
from jax.experimental.pallas import tpu_sc as plsc

N = 100000
E = 3200000
HID = 16

NPAD = 102400
NP8 = NPAD // 8
NREP = NPAD * HID

NC = 2
NS = 16
NW = NC * NS
G = 128
EROWS = 25600
KJ = 5
NB = 4
RB = 2
RA0 = 1100
RA1 = 500
RD0 = 1100
RD1 = 500

_mesh = plsc.VectorSubcoreMesh(core_axis_name="c", subcore_axis_name="s")
_sc_params = pltpu.CompilerParams(use_tc_tiling_on_sc=False)


def _fill_rows(buf, nrows, vec):
  def body(i, _):
    buf[i] = vec
    return 0
  lax.fori_loop(0, nrows, body, 0)


def _zero_acc(acc, sid, zrows):
  chunk = NPAD // NS
  def zb(i, _):
    pltpu.sync_copy(zrows, acc.at[pl.ds(sid * chunk + i * 128, 128)])
    return 0
  lax.fori_loop(0, chunk // 128, zb, 0)


def _copy_out(acc, out_hbm, cid, sid):
  chunk = NPAD // NS
  off = sid * chunk
  pltpu.sync_copy(acc.at[pl.ds(off, chunk)], out_hbm.at[cid, pl.ds(off, chunk)])


@functools.partial(
    pl.kernel,
    mesh=_mesh,
    compiler_params=_sc_params,
    out_type=jax.ShapeDtypeStruct((NC, NPAD, HID), jnp.float32),
    scratch_types=[
        pltpu.VMEM((NB, KJ, G), jnp.int32),
        pltpu.VMEM((NB, KJ, G), jnp.int32),
        pltpu.VMEM((G, HID), jnp.float32),
        pltpu.VMEM((G, HID), jnp.float32),
        pltpu.VMEM((128, HID), jnp.float32),
        pltpu.VMEM_SHARED((NPAD, HID), jnp.float32),
        pltpu.SemaphoreType.DMA,
        pltpu.SemaphoreType.DMA,
    ],
)
def _sc_degrees(src_hbm, dst_hbm, out_hbm, idxS, idxD, e0buf, e1buf, zrows, acc,
                sem_i, sem_s):
  cid = lax.axis_index("c")
  sid = lax.axis_index("s")

  lane = lax.iota(jnp.int32, HID)
  zvec = jnp.zeros((HID,), jnp.float32)
  e0 = jnp.where(lane == 0, 1.0, 0.0).astype(jnp.float32)
  e1 = jnp.where(lane == 1, 1.0, 0.0).astype(jnp.float32)
  _fill_rows(e0buf, G, e0)
  _fill_rows(e1buf, G, e1)
  _fill_rows(zrows, 128, zvec)

  _zero_acc(acc, sid, zrows)
  plsc.subcore_barrier()

  def pipeline(r0, outer):
    def idx_copies(g, b):
      r = r0 + g * KJ
      return (pltpu.make_async_copy(src_hbm.at[pl.ds(r, KJ)], idxS.at[b], sem_i),
              pltpu.make_async_copy(dst_hbm.at[pl.ds(r, KJ)], idxD.at[b], sem_i))

    def scat_copies(b, j):
      return (pltpu.make_async_copy(e0buf, acc.at[idxS.at[b, j]], sem_s),
              pltpu.make_async_copy(e1buf, acc.at[idxD.at[b, j]], sem_s))

    for g in range(min(2, outer)):
      a, c = idx_copies(g, g % NB)
      a.start(); c.start()

    def body(g, _):
      b = g % NB
      a, c = idx_copies(g, b)
      a.wait(); c.wait()
      for j in range(KJ):
        a, c = scat_copies(b, j)
        a.start(add=True); c.start(add=True)
      @pl.when(g + 2 < outer)
      def _():
        b2 = (g + 2) % NB
        @pl.when(g >= 2)
        def _():
          for j in range(KJ):
            a, c = scat_copies(b2, j)
            a.wait(); c.wait()
        a, c = idx_copies(g + 2, b2)
        a.start(); c.start()
      return 0
    lax.fori_loop(0, outer, body, 0)

    for g in range(max(0, outer - NB), outer):
      for j in range(KJ):
        a, c = scat_copies(g % NB, j)
        a.wait(); c.wait()

  @pl.when(cid == 0)
  def _():
    pipeline(sid * RD0, RD0 // KJ)
  @pl.when(cid == 1)
  def _():
    pipeline(16 * RD0 + sid * RD1, RD1 // KJ)
  plsc.subcore_barrier()

  _copy_out(acc, out_hbm, cid, sid)


@functools.partial(
    pl.kernel,
    mesh=_mesh,
    compiler_params=_sc_params,
    out_type=jax.ShapeDtypeStruct((NC, NPAD, HID), jnp.float32),
    scratch_types=[
        pltpu.VMEM((NB, KJ, G), jnp.int32),
        pltpu.VMEM((NB, KJ, G), jnp.int32),
        pltpu.VMEM((RB, KJ, G, HID), jnp.float32),
        pltpu.VMEM((128, HID), jnp.float32),
        pltpu.VMEM_SHARED((NPAD, HID), jnp.float32),
        pltpu.SemaphoreType.DMA,
        pltpu.SemaphoreType.DMA,
        pltpu.SemaphoreType.DMA,
    ],
)
def _sc_agg_vec(h_hbm, src_hbm, dst_hbm, out_hbm,
                idxS, idxD, rows, zrows, acc, sem_i, sem_g, sem_s):
  cid = lax.axis_index("c")
  sid = lax.axis_index("s")

  _fill_rows(zrows, 128, jnp.zeros((HID,), jnp.float32))
  _zero_acc(acc, sid, zrows)
  plsc.subcore_barrier()

  def pipeline(r0, outer):
    def idx_copies(g, b):
      r = r0 + g * KJ
      return (pltpu.make_async_copy(src_hbm.at[pl.ds(r, KJ)], idxS.at[b], sem_i),
              pltpu.make_async_copy(dst_hbm.at[pl.ds(r, KJ)], idxD.at[b], sem_i))

    def gath_copy(b, rb, j):
      return pltpu.make_async_copy(h_hbm.at[idxS.at[b, j]], rows.at[rb, j], sem_g)

    def scat_copy(b, rb, j):
      return pltpu.make_async_copy(rows.at[rb, j], acc.at[idxD.at[b, j]], sem_s)

    for g in range(min(2, outer)):
      a, c = idx_copies(g, g % NB)
      a.start(); c.start()

    def body(g, _):
      b = g % NB
      rb = g % RB
      a, c = idx_copies(g, b)
      a.wait(); c.wait()
      @pl.when(g >= 2)
      def _():
        for j in range(KJ):
          scat_copy((g + 2) % NB, rb, j).wait()
      for j in range(KJ):
        gath_copy(b, rb, j).start()
      @pl.when(g + 2 < outer)
      def _():
        a, c = idx_copies(g + 2, (g + 2) % NB)
        a.start(); c.start()
      for j in range(KJ):
        gath_copy(b, rb, j).wait()
      for j in range(KJ):
        scat_copy(b, rb, j).start(add=True)
      return 0
    lax.fori_loop(0, outer, body, 0)

    for g in range(max(0, outer - 2), outer):
      for j in range(KJ):
        scat_copy(g % NB, g % RB, j).wait()

  @pl.when(cid == 0)
  def _():
    pipeline(sid * RA0, RA0 // KJ)
  @pl.when(cid == 1)
  def _():
    pipeline(16 * RA0 + sid * RA1, RA1 // KJ)
  plsc.subcore_barrier()

  _copy_out(acc, out_hbm, cid, sid)


BP8 = 1280
GP8 = NP8 // BP8


def _p8_spec():
  return pl.BlockSpec((BP8, 128), lambda i: (i, 0))


def _p8hi_spec():
  return pl.BlockSpec((BP8, 128), lambda i: (i + GP8, 0))


def _w_spec():
  return pl.BlockSpec((128, 128), lambda i: (0, 0))


def _r_spec():
  return pl.BlockSpec((1, 128), lambda i: (0, 0))


def _tc_norms_body(d0, d1, xr, s0, s1, w0, nsr, ndr, h0):
  d = d0[...] + d1[...]
  od = lax.dot_general(d, s0[...], (((1,), (0,)), ((), ())),
                       preferred_element_type=jnp.float32)
  idg = lax.dot_general(d, s1[...], (((1,), (0,)), ((), ())),
                        preferred_element_type=jnp.float32)
  ns_v = lax.rsqrt(jnp.maximum(od, 1.0))
  nsr[...] = ns_v
  ndr[...] = lax.rsqrt(jnp.maximum(idg, 1.0))
  h0[...] = xr[...] * ns_v * w0[...]


def _tc_norms(degv, x_rep, s0, s1, w0_big):
  return pl.pallas_call(
      _tc_norms_body,
      grid=(GP8,),
      in_specs=[_p8_spec(), _p8hi_spec(), _p8_spec(), _w_spec(), _w_spec(),
                _r_spec()],
      out_specs=[_p8_spec()] * 3,
      out_shape=[jax.ShapeDtypeStruct((NP8, 128), jnp.float32)] * 3,
  )(degv, degv, x_rep, s0, s1, w0_big)


def _tc_affine_body(a0, a1, ndr, nsr, b, out):
  t = (a0[...] + a1[...]) * ndr[...] + b[...]
  out[...] = jnp.maximum(t, 0.0) * nsr[...]


def _tc_affine(aggv, ndr, nsr, b_big):
  return pl.pallas_call(
      _tc_affine_body,
      grid=(GP8,),
      in_specs=[_p8_spec(), _p8hi_spec(), _p8_spec(), _p8_spec(), _r_spec()],
      out_specs=_p8_spec(),
      out_shape=jax.ShapeDtypeStruct((NP8, 128), jnp.float32),
  )(aggv, aggv, ndr, nsr, b_big)


def _make_tc_dense(relu, scale_src):
  def body(a0, a1, ndr, nsr, w, b, out):
    t = (a0[...] + a1[...]) * ndr[...]
    h = lax.dot_general(t, w[...], (((1,), (0,)), ((), ())),
                        preferred_element_type=jnp.float32) + b[...]
    if relu:
      h = jnp.maximum(h, 0.0)
    if scale_src:
      h = h * nsr[...]
    out[...] = h

  def run(aggv, ndr, nsr, w_big, b_big):
    return pl.pallas_call(
        body,
        grid=(GP8,),
        in_specs=[_p8_spec(), _p8hi_spec(), _p8_spec(), _p8_spec(),
                  _w_spec(), _r_spec()],
        out_specs=_p8_spec(),
        out_shape=jax.ShapeDtypeStruct((NP8, 128), jnp.float32),
    )(aggv, aggv, ndr, nsr, w_big, b_big)
  return run


_tc_dense_mid = _make_tc_dense(relu=True, scale_src=True)
_tc_dense_last = _make_tc_dense(relu=False, scale_src=False)


def _flat(agg):
  return agg.reshape(2 * NP8, 128)


@jax.jit
def kernel(x, edge_index, W0, b0, W1, b1, W2, b2):
  src = edge_index[0]
  dst = edge_index[1]
  pad = EROWS * G - E
  src_p = jnp.concatenate([src, jnp.full((pad,), NPAD - 1, jnp.int32)]).reshape(EROWS, G)
  dst_p = jnp.concatenate([dst, jnp.full((pad,), NPAD - 1, jnp.int32)]).reshape(EROWS, G)

  x_pad = jnp.concatenate([x.reshape(N), jnp.zeros((NPAD - N,), jnp.float32)])
  x_rep = jnp.repeat(x_pad, HID).reshape(NP8, 128)
  ii = jnp.arange(128)
  base = (ii[None, :] // HID) * HID
  s0 = (ii[:, None] == base).astype(jnp.float32)
  s1 = (ii[:, None] == base + 1).astype(jnp.float32)

  w0_big = jnp.tile(W0.reshape(HID), 8).reshape(1, 128)
  b0_big = jnp.tile(b0, 8).reshape(1, 128)
  w1_big = jnp.kron(jnp.eye(8, dtype=jnp.float32), W1)
  b1_big = jnp.tile(b1, 8).reshape(1, 128)
  w2_big = jnp.kron(jnp.eye(8, dtype=jnp.float32), W2)
  b2_big = jnp.tile(b2, 8).reshape(1, 128)

  deg_p = _sc_degrees(src_p, dst_p)
  ns_rep, nd_rep, h0 = _tc_norms(_flat(deg_p), x_rep, s0, s1, w0_big)

  agg1 = _sc_agg_vec(h0.reshape(NPAD, HID), src_p, dst_p)
  h1 = _tc_affine(_flat(agg1), nd_rep, ns_rep, b0_big)

  agg2 = _sc_agg_vec(h1.reshape(NPAD, HID), src_p, dst_p)
  h2 = _tc_dense_mid(_flat(agg2), nd_rep, ns_rep, w1_big, b1_big)

  agg3 = _sc_agg_vec(h2.reshape(NPAD, HID), src_p, dst_p)
  out = _tc_dense_last(_flat(agg3), nd_rep, ns_rep, w2_big, b2_big)
  return out.reshape(NPAD, HID)[:N]


# speedup vs baseline: 2.6748x; 1.7149x over previous
import functools

import jax
import jax.numpy as jnp
from jax import lax
from jax.experimental import pallas as pl
from jax.experimental.pallas import tpu as pltpu
from jax.experimental.pallas import tpu_sc as plsc

N = 100000
E = 3200000
HID = 16

NPAD = 102400
NP8 = NPAD // 8
NREP = NPAD * HID

NC = 2
NS = 16
NW = NC * NS
G = 128
EROWS = 25000
KJ = 5
NB = 4
RB = 2
RA0 = 1080
RA1 = 480
RD0 = 1080
RD1 = 480
REM_BASE = 16 * (RA0 + RA1)

_mesh = plsc.VectorSubcoreMesh(core_axis_name="c", subcore_axis_name="s")
_sc_params = pltpu.CompilerParams(use_tc_tiling_on_sc=False)


def _fill_rows(buf, nrows, vec):
  def body(i, _):
    buf[i] = vec
    return 0
  lax.fori_loop(0, nrows, body, 0)


def _zero_acc(acc, sid, zrows):
  chunk = NPAD // NS
  def zb(i, _):
    pltpu.sync_copy(zrows, acc.at[pl.ds(sid * chunk + i * 128, 128)])
    return 0
  lax.fori_loop(0, chunk // 128, zb, 0)


def _copy_out(acc, out_hbm, cid, sid):
  chunk = NPAD // NS
  off = sid * chunk
  pltpu.sync_copy(acc.at[pl.ds(off, chunk)], out_hbm.at[cid, pl.ds(off, chunk)])


@functools.partial(
    pl.kernel,
    mesh=_mesh,
    compiler_params=_sc_params,
    out_type=jax.ShapeDtypeStruct((NC, NPAD, HID), jnp.float32),
    scratch_types=[
        pltpu.VMEM((NB, KJ, G), jnp.int32),
        pltpu.VMEM((NB, KJ, G), jnp.int32),
        pltpu.VMEM((G, HID), jnp.float32),
        pltpu.VMEM((G, HID), jnp.float32),
        pltpu.VMEM((128, HID), jnp.float32),
        pltpu.VMEM_SHARED((NPAD, HID), jnp.float32),
        pltpu.SemaphoreType.DMA,
        pltpu.SemaphoreType.DMA,
    ],
)
def _sc_degrees(ei_hbm, out_hbm, idxS, idxD, e0buf, e1buf, zrows, acc,
                sem_i, sem_s):
  cid = lax.axis_index("c")
  sid = lax.axis_index("s")

  lane = lax.iota(jnp.int32, HID)
  zvec = jnp.zeros((HID,), jnp.float32)
  e0 = jnp.where(lane == 0, 1.0, 0.0).astype(jnp.float32)
  e1 = jnp.where(lane == 1, 1.0, 0.0).astype(jnp.float32)
  _fill_rows(e0buf, G, e0)
  _fill_rows(e1buf, G, e1)
  _fill_rows(zrows, 128, zvec)

  _zero_acc(acc, sid, zrows)
  plsc.subcore_barrier()

  def pipeline(r0, outer):
    def idx_copies(g, b):
      r = r0 + g * KJ
      return (pltpu.make_async_copy(ei_hbm.at[0, pl.ds(r, KJ)], idxS.at[b], sem_i),
              pltpu.make_async_copy(ei_hbm.at[1, pl.ds(r, KJ)], idxD.at[b], sem_i))

    def scat_copies(b, j):
      return (pltpu.make_async_copy(e0buf, acc.at[idxS.at[b, j]], sem_s),
              pltpu.make_async_copy(e1buf, acc.at[idxD.at[b, j]], sem_s))

    for g in range(min(2, outer)):
      a, c = idx_copies(g, g % NB)
      a.start(); c.start()

    def body(g, _):
      b = g % NB
      a, c = idx_copies(g, b)
      a.wait(); c.wait()
      for j in range(KJ):
        a, c = scat_copies(b, j)
        a.start(add=True); c.start(add=True)
      @pl.when(g + 2 < outer)
      def _():
        b2 = (g + 2) % NB
        @pl.when(g >= 2)
        def _():
          for j in range(KJ):
            a, c = scat_copies(b2, j)
            a.wait(); c.wait()
        a, c = idx_copies(g + 2, b2)
        a.start(); c.start()
      return 0
    lax.fori_loop(0, outer, body, 0)

    for g in range(max(0, outer - NB), outer):
      for j in range(KJ):
        a, c = scat_copies(g % NB, j)
        a.wait(); c.wait()

  @pl.when(cid == 0)
  def _():
    pipeline(sid * RD0, RD0 // KJ)
  @pl.when(cid == 1)
  def _():
    pipeline(16 * RD0 + sid * RD1, RD1 // KJ)
  w = sid * NC + cid
  @pl.when(w < (EROWS - REM_BASE) // KJ)
  def _():
    pipeline(REM_BASE + w * KJ, 1)
  plsc.subcore_barrier()

  _copy_out(acc, out_hbm, cid, sid)


@functools.partial(
    pl.kernel,
    mesh=_mesh,
    compiler_params=_sc_params,
    out_type=jax.ShapeDtypeStruct((NC, NPAD, HID), jnp.float32),
    scratch_types=[
        pltpu.VMEM((NB, KJ, G), jnp.int32),
        pltpu.VMEM((NB, KJ, G), jnp.int32),
        pltpu.VMEM((RB, KJ, G, HID), jnp.float32),
        pltpu.VMEM((128, HID), jnp.float32),
        pltpu.VMEM_SHARED((NPAD, HID), jnp.float32),
        pltpu.SemaphoreType.DMA,
        pltpu.SemaphoreType.DMA,
        pltpu.SemaphoreType.DMA,
    ],
)
def _sc_agg_vec(h_hbm, ei_hbm, out_hbm,
                idxS, idxD, rows, zrows, acc, sem_i, sem_g, sem_s):
  cid = lax.axis_index("c")
  sid = lax.axis_index("s")

  _fill_rows(zrows, 128, jnp.zeros((HID,), jnp.float32))
  _zero_acc(acc, sid, zrows)
  plsc.subcore_barrier()

  def pipeline(r0, outer):
    def idx_copies(g, b):
      r = r0 + g * KJ
      return (pltpu.make_async_copy(ei_hbm.at[0, pl.ds(r, KJ)], idxS.at[b], sem_i),
              pltpu.make_async_copy(ei_hbm.at[1, pl.ds(r, KJ)], idxD.at[b], sem_i))

    def gath_copy(b, rb, j):
      return pltpu.make_async_copy(h_hbm.at[idxS.at[b, j]], rows.at[rb, j], sem_g)

    def scat_copy(b, rb, j):
      return pltpu.make_async_copy(rows.at[rb, j], acc.at[idxD.at[b, j]], sem_s)

    for g in range(min(2, outer)):
      a, c = idx_copies(g, g % NB)
      a.start(); c.start()

    def body(g, _):
      b = g % NB
      rb = g % RB
      a, c = idx_copies(g, b)
      a.wait(); c.wait()
      @pl.when(g >= 2)
      def _():
        for j in range(KJ):
          scat_copy((g + 2) % NB, rb, j).wait()
      for j in range(KJ):
        gath_copy(b, rb, j).start()
      @pl.when(g + 2 < outer)
      def _():
        a, c = idx_copies(g + 2, (g + 2) % NB)
        a.start(); c.start()
      for j in range(KJ):
        gath_copy(b, rb, j).wait()
      for j in range(KJ):
        scat_copy(b, rb, j).start(add=True)
      return 0
    lax.fori_loop(0, outer, body, 0)

    for g in range(max(0, outer - 2), outer):
      for j in range(KJ):
        scat_copy(g % NB, g % RB, j).wait()

  @pl.when(cid == 0)
  def _():
    pipeline(sid * RA0, RA0 // KJ)
  @pl.when(cid == 1)
  def _():
    pipeline(16 * RA0 + sid * RA1, RA1 // KJ)
  w = sid * NC + cid
  @pl.when(w < (EROWS - REM_BASE) // KJ)
  def _():
    pipeline(REM_BASE + w * KJ, 1)
  plsc.subcore_barrier()

  _copy_out(acc, out_hbm, cid, sid)


BP8 = 1280
GP8 = NP8 // BP8


def _p8_spec():
  return pl.BlockSpec((BP8, 128), lambda i: (i, 0))


def _p8hi_spec():
  return pl.BlockSpec((BP8, 128), lambda i: (i + GP8, 0))


def _w_spec():
  return pl.BlockSpec((128, 128), lambda i: (0, 0))


def _r_spec():
  return pl.BlockSpec((1, 128), lambda i: (0, 0))


def _tc_norms_body(d0, d1, xr, s0, s1, w0, nsr, ndr, h0):
  d = d0[...] + d1[...]
  od = lax.dot_general(d, s0[...], (((1,), (0,)), ((), ())),
                       preferred_element_type=jnp.float32)
  idg = lax.dot_general(d, s1[...], (((1,), (0,)), ((), ())),
                        preferred_element_type=jnp.float32)
  ns_v = lax.rsqrt(jnp.maximum(od, 1.0))
  nsr[...] = ns_v
  ndr[...] = lax.rsqrt(jnp.maximum(idg, 1.0))
  h0[...] = xr[...] * ns_v * w0[...]


def _tc_norms(degv, x_rep, s0, s1, w0_big):
  return pl.pallas_call(
      _tc_norms_body,
      grid=(GP8,),
      in_specs=[_p8_spec(), _p8hi_spec(), _p8_spec(), _w_spec(), _w_spec(),
                _r_spec()],
      out_specs=[_p8_spec()] * 3,
      out_shape=[jax.ShapeDtypeStruct((NP8, 128), jnp.float32)] * 3,
  )(degv, degv, x_rep, s0, s1, w0_big)


def _tc_affine_body(a0, a1, ndr, nsr, b, out):
  t = (a0[...] + a1[...]) * ndr[...] + b[...]
  out[...] = jnp.maximum(t, 0.0) * nsr[...]


def _tc_affine(aggv, ndr, nsr, b_big):
  return pl.pallas_call(
      _tc_affine_body,
      grid=(GP8,),
      in_specs=[_p8_spec(), _p8hi_spec(), _p8_spec(), _p8_spec(), _r_spec()],
      out_specs=_p8_spec(),
      out_shape=jax.ShapeDtypeStruct((NP8, 128), jnp.float32),
  )(aggv, aggv, ndr, nsr, b_big)


def _make_tc_dense(relu, scale_src):
  def body(a0, a1, ndr, nsr, w, b, out):
    t = (a0[...] + a1[...]) * ndr[...]
    h = lax.dot_general(t, w[...], (((1,), (0,)), ((), ())),
                        preferred_element_type=jnp.float32) + b[...]
    if relu:
      h = jnp.maximum(h, 0.0)
    if scale_src:
      h = h * nsr[...]
    out[...] = h

  def run(aggv, ndr, nsr, w_big, b_big):
    return pl.pallas_call(
        body,
        grid=(GP8,),
        in_specs=[_p8_spec(), _p8hi_spec(), _p8_spec(), _p8_spec(),
                  _w_spec(), _r_spec()],
        out_specs=_p8_spec(),
        out_shape=jax.ShapeDtypeStruct((NP8, 128), jnp.float32),
    )(aggv, aggv, ndr, nsr, w_big, b_big)
  return run


_tc_dense_mid = _make_tc_dense(relu=True, scale_src=True)
_tc_dense_last = _make_tc_dense(relu=False, scale_src=False)


def _flat(agg):
  return agg.reshape(2 * NP8, 128)


@jax.jit
def kernel(x, edge_index, W0, b0, W1, b1, W2, b2):
  ei3 = edge_index.reshape(2, EROWS, G)

  x_pad = jnp.concatenate([x.reshape(N), jnp.zeros((NPAD - N,), jnp.float32)])
  x_rep = jnp.repeat(x_pad, HID).reshape(NP8, 128)
  ii = jnp.arange(128)
  base = (ii[None, :] // HID) * HID
  s0 = (ii[:, None] == base).astype(jnp.float32)
  s1 = (ii[:, None] == base + 1).astype(jnp.float32)

  w0_big = jnp.tile(W0.reshape(HID), 8).reshape(1, 128)
  b0_big = jnp.tile(b0, 8).reshape(1, 128)
  w1_big = jnp.kron(jnp.eye(8, dtype=jnp.float32), W1)
  b1_big = jnp.tile(b1, 8).reshape(1, 128)
  w2_big = jnp.kron(jnp.eye(8, dtype=jnp.float32), W2)
  b2_big = jnp.tile(b2, 8).reshape(1, 128)

  deg_p = _sc_degrees(ei3)
  ns_rep, nd_rep, h0 = _tc_norms(_flat(deg_p), x_rep, s0, s1, w0_big)

  agg1 = _sc_agg_vec(h0.reshape(NPAD, HID), ei3)
  h1 = _tc_affine(_flat(agg1), nd_rep, ns_rep, b0_big)

  agg2 = _sc_agg_vec(h1.reshape(NPAD, HID), ei3)
  h2 = _tc_dense_mid(_flat(agg2), nd_rep, ns_rep, w1_big, b1_big)

  agg3 = _sc_agg_vec(h2.reshape(NPAD, HID), ei3)
  out = _tc_dense_last(_flat(agg3), nd_rep, ns_rep, w2_big, b2_big)
  return out.reshape(NPAD, HID)[:N]
